# baseline (speedup 1.0000x reference)
import functools

import jax
import jax.numpy as jnp
from jax.experimental import pallas as pl
from jax.experimental.pallas import tpu as pltpu

_ONES_W = 128


def _hypersage_kernel(
    a_ref, x_ref, w1_ref, w2_ref, out_ref,
    a16_scr, s_scr, m_scr, invn_scr,
    *, nsteps, bn, d,
):
    ph = pl.program_id(0)
    i = pl.program_id(1)
    last = nsteps - 1

    @pl.when(ph == 0)
    def _phase0():
        a16 = a_ref[...].astype(jnp.bfloat16)
        a16_scr[i] = a16
        y = x_ref[...]
        y2 = (y * y).astype(jnp.bfloat16)
        ones_rows = jnp.ones((bn, _ONES_W), jnp.bfloat16)
        y_aug = jnp.concatenate([y2, ones_rows], axis=1)
        part = jax.lax.dot_general(
            y_aug, a16, (((0,), (0,)), ((), ())),
            preferred_element_type=jnp.float32,
        )

        @pl.when(i == 0)
        def _init():
            s_scr[...] = part

        @pl.when(i > 0)
        def _acc():
            s_scr[...] += part

        @pl.when(i == last)
        def _finish():
            invn = 1.0 / jnp.maximum(s_scr[d : d + 1, :], 1.0)
            invn_scr[...] = invn
            m_scr[:, :d] = jnp.transpose((s_scr[:d, :] * invn).astype(jnp.bfloat16))
            m_scr[:, d:] = jnp.ones((m_scr.shape[0], _ONES_W), jnp.bfloat16)

    @pl.when(ph == 1)
    def _phase1():
        a16 = a16_scr[i]
        z_aug = jnp.dot(a16, m_scr[...], preferred_element_type=jnp.float32)
        inve = 1.0 / jnp.maximum(z_aug[:, d : d + 1], 1.0)
        u = jnp.sqrt(z_aug[:, :d] * inve)
        h = jnp.maximum(
            jnp.dot(u.astype(jnp.bfloat16), w1_ref[...],
                    preferred_element_type=jnp.float32),
            0.0,
        )
        h2 = (h * h).astype(jnp.bfloat16)
        part = jax.lax.dot_general(
            h2, a16, (((0,), (0,)), ((), ())),
            preferred_element_type=jnp.float32,
        )

        @pl.when(i == 0)
        def _init():
            s_scr[:d, :] = part

        @pl.when(i > 0)
        def _acc():
            s_scr[:d, :] += part

        @pl.when(i == last)
        def _finish():
            m_scr[:, :d] = jnp.transpose(
                (s_scr[:d, :] * invn_scr[...]).astype(jnp.bfloat16)
            )

    @pl.when(ph == 2)
    def _phase2():
        a16 = a16_scr[i]
        z_aug = jnp.dot(a16, m_scr[...], preferred_element_type=jnp.float32)
        inve = 1.0 / jnp.maximum(z_aug[:, d : d + 1], 1.0)
        u = jnp.sqrt(z_aug[:, :d] * inve)
        out_ref[...] = jnp.maximum(
            jnp.dot(u.astype(jnp.bfloat16), w2_ref[...],
                    preferred_element_type=jnp.float32),
            0.0,
        )


def kernel(x_0, incidence, W1, W2):
    n, d = x_0.shape
    e = incidence.shape[1]
    bn = 400
    nsteps = n // bn

    w1_16 = W1.astype(jnp.bfloat16)
    w2_16 = W2.astype(jnp.bfloat16)

    out = pl.pallas_call(
        functools.partial(_hypersage_kernel, nsteps=nsteps, bn=bn, d=d),
        grid=(3, nsteps),
        in_specs=[
            pl.BlockSpec((bn, e), lambda ph, i: (jnp.where(ph == 0, i, 0), 0)),
            pl.BlockSpec((bn, d), lambda ph, i: (jnp.where(ph == 0, i, 0), 0)),
            pl.BlockSpec((d, d), lambda ph, i: (0, 0)),
            pl.BlockSpec((d, d), lambda ph, i: (0, 0)),
        ],
        out_specs=pl.BlockSpec((bn, d), lambda ph, i: (jnp.where(ph == 2, i, 0), 0)),
        out_shape=jax.ShapeDtypeStruct((n, d), jnp.float32),
        scratch_shapes=[
            pltpu.VMEM((nsteps, bn, e), jnp.bfloat16),
            pltpu.VMEM((d + _ONES_W, e), jnp.float32),
            pltpu.VMEM((e, d + _ONES_W), jnp.bfloat16),
            pltpu.VMEM((1, e), jnp.float32),
        ],
        compiler_params=pltpu.CompilerParams(vmem_limit_bytes=112 * 1024 * 1024),
    )(incidence, x_0, w1_16, w2_16)

    return out
